# Initial kernel scaffold; baseline (speedup 1.0000x reference)
import jax
import jax.numpy as jnp
from jax import lax
from jax.experimental import pallas as pl
from jax.experimental.pallas import tpu as pltpu
from jax.experimental.pallas import tpu_sc as plsc

NC = 2
NS = 16
NW = NC * NS
L = 16


def _pick_block(ew: int) -> int:
    best = L
    for b in range(L, 4097, L):
        if ew % b == 0:
            best = b
    return best


def kernel(charges, cell, positions, neighbor_indices, neighbor_distances):
    n, c = charges.shape
    e = neighbor_indices.shape[0]
    assert e % NW == 0, e
    ew = e // NW
    blk = _pick_block(ew)
    nblk = ew // blk

    mesh = plsc.VectorSubcoreMesh(core_axis_name="c", subcore_axis_name="s")

    def body(q_hbm, nidx_hbm, ndist_hbm, zeros_hbm, out_hbm,
             q_sp, acc_sp, idx_blk, w_blk, idxi, idxj, rows_a, rows_b):
        cid = lax.axis_index("c")
        sid = lax.axis_index("s")
        wid = cid * NS + sid

        @pl.when(sid == 0)
        def _stage():
            pltpu.sync_copy(q_hbm, q_sp)
            pltpu.sync_copy(zeros_hbm, acc_sp)

        plsc.subcore_barrier()

        iota = lax.iota(jnp.int32, L)
        pat = iota // c
        colpat = iota - pat * c
        col0 = iota * 0
        col1 = col0 + 1
        base = wid * ew

        def block(b, carry):
            off = base + b * blk
            pltpu.sync_copy(nidx_hbm.at[pl.ds(off, blk)], idx_blk)
            pltpu.sync_copy(ndist_hbm.at[pl.ds(off, blk)], w_blk)

            def grp(g, carry2):
                s = g * L
                e16 = s + iota
                idxi[pl.ds(s, L)] = plsc.load_gather(idx_blk, [e16, col0])
                idxj[pl.ds(s, L)] = plsc.load_gather(idx_blk, [e16, col1])
                w_blk[pl.ds(s, L)] = 0.5 / w_blk[pl.ds(s, L)]
                return carry2

            lax.fori_loop(0, blk // L, grp, 0)

            pltpu.sync_copy(q_sp.at[idxj], rows_a)
            pltpu.sync_copy(q_sp.at[idxi], rows_b)

            def sgrp(g, carry2):
                r = g * (L // c)
                row_idx = r + pat
                w16 = plsc.load_gather(w_blk, [row_idx])
                va = plsc.load_gather(rows_a, [row_idx, colpat]) * w16
                plsc.store_scatter(rows_a, [row_idx, colpat], va)
                vb = plsc.load_gather(rows_b, [row_idx, colpat]) * w16
                plsc.store_scatter(rows_b, [row_idx, colpat], vb)
                return carry2

            lax.fori_loop(0, (blk * c) // L, sgrp, 0)

            pltpu.sync_copy(rows_a, acc_sp.at[idxi], add=True)
            pltpu.sync_copy(rows_b, acc_sp.at[idxj], add=True)
            return carry

        lax.fori_loop(0, nblk, block, 0)

        plsc.subcore_barrier()

        @pl.when(sid == 0)
        def _writeout():
            pltpu.sync_copy(acc_sp, out_hbm.at[pl.ds(cid * n, n)])

    kfn = pl.kernel(
        body,
        out_type=jax.ShapeDtypeStruct((NC * n, c), jnp.float32),
        mesh=mesh,
        scratch_types=[
            pltpu.VMEM_SHARED((n, c), jnp.float32),
            pltpu.VMEM_SHARED((n, c), jnp.float32),
            pltpu.VMEM((blk, 2), jnp.int32),
            pltpu.VMEM((blk,), jnp.float32),
            pltpu.VMEM((blk,), jnp.int32),
            pltpu.VMEM((blk,), jnp.int32),
            pltpu.VMEM((blk, c), jnp.float32),
            pltpu.VMEM((blk, c), jnp.float32),
        ],
    )

    zeros = jnp.zeros((n, c), jnp.float32)
    partial = kfn(charges, neighbor_indices, neighbor_distances, zeros)
    return partial[:n] + partial[n:]


# speedup vs baseline: 7.9714x; 7.9714x over previous
import jax
import jax.numpy as jnp
from jax import lax
from jax.experimental import pallas as pl
from jax.experimental.pallas import tpu as pltpu
from jax.experimental.pallas import tpu_sc as plsc

NC = 2
NS = 16
NW = NC * NS
L = 16
CP = 8


def _pick_block(ew: int) -> int:
    best = L
    for b in range(L, 1001, L):
        if ew % b == 0:
            best = b
    return best


def kernel(charges, cell, positions, neighbor_indices, neighbor_distances):
    n, c = charges.shape
    e = neighbor_indices.shape[0]
    assert e % NW == 0, e
    ew = e // NW
    blk = _pick_block(ew)
    nblk = ew // blk

    mesh = plsc.VectorSubcoreMesh(
        core_axis_name="c", subcore_axis_name="s", num_cores=NC, num_subcores=NS)

    def body(q_hbm, nidx_hbm, ndist_hbm, zeros_hbm, out_hbm,
             q_sp, acc_sp, idx_blk, w_blk, idxi, idxj, rows_a, rows_b):
        cid = lax.axis_index("c")
        sid = lax.axis_index("s")
        wid = cid * NS + sid

        @pl.when(sid == 0)
        def _stage():
            pltpu.sync_copy(q_hbm, q_sp)
            pltpu.sync_copy(zeros_hbm, acc_sp)

        plsc.subcore_barrier()

        iota = lax.iota(jnp.int32, L)
        pat = iota // CP
        colpat = iota - pat * CP
        base = wid * ew

        def block(b, carry):
            off = base + b * blk
            pltpu.sync_copy(nidx_hbm.at[pl.ds(2 * off, 2 * blk)], idx_blk)
            pltpu.sync_copy(ndist_hbm.at[pl.ds(off, blk)], w_blk)

            def grp(g, carry2):
                s = g * L
                e2 = 2 * (s + iota)
                idxi[pl.ds(s, L)] = plsc.load_gather(idx_blk, [e2])
                idxj[pl.ds(s, L)] = plsc.load_gather(idx_blk, [e2 + 1])
                w_blk[pl.ds(s, L)] = 0.5 / w_blk[pl.ds(s, L)]
                return carry2

            lax.fori_loop(0, blk // L, grp, 0)

            pltpu.sync_copy(q_sp.at[idxj], rows_a)
            pltpu.sync_copy(q_sp.at[idxi], rows_b)

            def sgrp(g, carry2):
                r = g * (L // CP)
                row_idx = r + pat
                w16 = plsc.load_gather(w_blk, [row_idx])
                va = plsc.load_gather(rows_a, [row_idx, colpat]) * w16
                plsc.store_scatter(rows_a, [row_idx, colpat], va)
                vb = plsc.load_gather(rows_b, [row_idx, colpat]) * w16
                plsc.store_scatter(rows_b, [row_idx, colpat], vb)
                return carry2

            lax.fori_loop(0, (blk * CP) // L, sgrp, 0)

            pltpu.sync_copy(rows_a, acc_sp.at[idxi], add=True)
            pltpu.sync_copy(rows_b, acc_sp.at[idxj], add=True)
            return carry

        lax.fori_loop(0, nblk, block, 0)

        plsc.subcore_barrier()

        @pl.when(sid == 0)
        def _writeout():
            pltpu.sync_copy(acc_sp, out_hbm.at[pl.ds(cid * n, n)])

    kfn = pl.kernel(
        body,
        out_type=jax.ShapeDtypeStruct((NC * n, CP), jnp.float32),
        mesh=mesh,
        compiler_params=pltpu.CompilerParams(
            needs_layout_passes=False, use_tc_tiling_on_sc=False),
        scratch_types=[
            pltpu.VMEM_SHARED((n, CP), jnp.float32),
            pltpu.VMEM_SHARED((n, CP), jnp.float32),
            pltpu.VMEM((2 * blk,), jnp.int32),
            pltpu.VMEM((blk,), jnp.float32),
            pltpu.VMEM((blk,), jnp.int32),
            pltpu.VMEM((blk,), jnp.int32),
            pltpu.VMEM((blk, CP), jnp.float32),
            pltpu.VMEM((blk, CP), jnp.float32),
        ],
    )

    qpad = jnp.pad(charges, ((0, 0), (0, CP - c)))
    zeros = jnp.zeros((n, CP), jnp.float32)
    partial = kfn(qpad, neighbor_indices.reshape(-1), neighbor_distances, zeros)
    return partial[:n, :c] + partial[n:, :c]
